# baseline (speedup 1.0000x reference)
import functools

import jax
import jax.numpy as jnp
from jax import lax
from jax.experimental import pallas as pl
from jax.experimental.pallas import tpu as pltpu
from jax.experimental.pallas import tpu_sc as plsc

N = 6400000
NUM_CH = 100000
NC = 2
NS = 16
NWK = NC * NS
PER_W = N // NWK
WIN = 4000
IDW = WIN + 32
NWIN = PER_W // WIN
UNROLL = 10
CHECK = 5
GPW = WIN // (16 * UNROLL)
LIST = 128
FLUSH_AT = LIST - 16 * CHECK
CH_SLICE = 6272
NUM_CH_PAD = NS * CH_SLICE


def _gather16(v, idx):
    return v.at[idx].get(mode="promise_in_bounds")


def _sc_body(out_hbm, tgt_hbm, ids_hbm, part_hbm,
             win_out, win_tgt, win_ids, stage,
             idx_l, sum_l, cmb_l,
             acc_sum, acc_cmb, dsem, fsem0, fsem1):
    c = lax.axis_index("c")
    s = lax.axis_index("s")
    wid = s * NC + c
    e0 = wid * PER_W
    io16 = lax.iota(jnp.int32, 16)
    io16p1 = io16 + 1
    m15 = io16 == 15
    pad_idx = NUM_CH + s * 16 + io16
    zero16 = jnp.zeros((16,), jnp.float32)
    fsems = (fsem0, fsem1)

    def _zb(i, _):
        stage[pl.ds(i * 16, 16)] = zero16
        return 0
    lax.fori_loop(0, CH_SLICE // 16, _zb, 0)
    off = s * CH_SLICE
    pltpu.sync_copy(stage, acc_sum.at[pl.ds(off, CH_SLICE)])
    pltpu.sync_copy(stage, acc_cmb.at[pl.ds(off, CH_SLICE)])

    def _fill(si):
        def _fb(k, _):
            idx_l[si, pl.ds(16 * k, 16)] = pad_idx
            sum_l[si, pl.ds(16 * k, 16)] = zero16
            cmb_l[si, pl.ds(16 * k, 16)] = zero16
            return 0
        lax.fori_loop(0, LIST // 16, _fb, 0)

    _fill(0)
    _fill(1)

    def _fire(si):
        pltpu.async_copy(sum_l.at[si], acc_sum.at[idx_l.at[si]], fsems[si], add=True)
        pltpu.async_copy(cmb_l.at[si], acc_cmb.at[idx_l.at[si]], fsems[si], add=True)

    def _drain(si):
        pltpu.make_async_copy(sum_l.at[si], acc_sum.at[idx_l.at[si]], fsems[si]).wait()
        pltpu.make_async_copy(cmb_l.at[si], acc_cmb.at[idx_l.at[si]], fsems[si]).wait()

    def _win_dma(w, slot):
        base = e0 + w * WIN
        pltpu.async_copy(out_hbm.at[pl.ds(base, WIN)],
                         win_out.at[pl.ds(slot * WIN, WIN)], dsem)
        pltpu.async_copy(tgt_hbm.at[pl.ds(base, WIN)],
                         win_tgt.at[pl.ds(slot * WIN, WIN)], dsem)
        pltpu.async_copy(ids_hbm.at[pl.ds(base - 16, WIN + 16)],
                         win_ids.at[pl.ds(slot * IDW, WIN + 16)], dsem)

    def _win_wait():
        pltpu.make_async_copy(out_hbm.at[pl.ds(0, WIN)],
                              win_out.at[pl.ds(0, WIN)], dsem).wait()
        pltpu.make_async_copy(tgt_hbm.at[pl.ds(0, WIN)],
                              win_tgt.at[pl.ds(0, WIN)], dsem).wait()
        pltpu.make_async_copy(ids_hbm.at[pl.ds(0, WIN + 16)],
                              win_ids.at[pl.ds(0, WIN + 16)], dsem).wait()

    @pl.when(wid > 0)
    def _():
        pltpu.async_copy(ids_hbm.at[pl.ds(e0 - 16, WIN + 16)],
                         win_ids.at[pl.ds(0, WIN + 16)], dsem)

    @pl.when(wid == 0)
    def _():
        win_ids[pl.ds(0, 16)] = jnp.full((16,), -1, jnp.int32)
        pltpu.async_copy(ids_hbm.at[pl.ds(0, WIN + 16)],
                         win_ids.at[pl.ds(16, WIN + 16)], dsem)

    pltpu.async_copy(out_hbm.at[pl.ds(e0, WIN)],
                     win_out.at[pl.ds(0, WIN)], dsem)
    pltpu.async_copy(tgt_hbm.at[pl.ds(e0, WIN)],
                     win_tgt.at[pl.ds(0, WIN)], dsem)

    def vec_body(ioff, boff, cursor, parity):
        ids = win_ids[pl.ds(ioff, 16)]
        sh = win_ids[pl.ds(ioff - 1, 16)]
        shl = win_ids[pl.ds(ioff + 1, 16)]
        x = win_out[pl.ds(boff, 16)]
        t = win_tgt[pl.ds(boff, 16)]
        p = 1.0 / (1.0 + jnp.exp(-x))
        sb = ids != sh
        r = plsc.cummax(jnp.where(sb, io16, 0))
        cs = plsc.cumsum(p)
        base = jnp.where(r == 0, 0.0,
                         _gather16(cs, jnp.maximum(r - 1, 0)))
        runsum = cs - base
        runcnt = (io16p1 - r).astype(jnp.float32)
        tstart = _gather16(jnp.where(sb, t, 0.0), r)
        comb = 2.0 * runcnt + tstart
        eb = (ids != shl) | m15
        plsc.store_compressed(idx_l.at[parity, pl.ds(cursor, 16)], ids, mask=eb)
        plsc.store_compressed(sum_l.at[parity, pl.ds(cursor, 16)], runsum, mask=eb)
        plsc.store_compressed(cmb_l.at[parity, pl.ds(cursor, 16)], comb, mask=eb)
        return cursor + jnp.sum(eb.astype(jnp.int32))

    def _check_flush(carry):
        cursor, parity, nflush = carry
        do_flush = cursor >= FLUSH_AT

        @pl.when(do_flush)
        def _():
            for si in range(2):
                @pl.when(parity == si)
                def _(si=si):
                    _fire(si)

                    @pl.when(nflush >= 1)
                    def _():
                        _drain(1 - si)
                        _fill(1 - si)

        cursor = jnp.where(do_flush, 0, cursor)
        parity = jnp.where(do_flush, 1 - parity, parity)
        nflush = jnp.where(do_flush, nflush + 1, nflush)
        return (cursor, parity, nflush)

    def grp_body(slot, g, carry):
        cursor, parity, nflush = carry
        ibase = slot * IDW + 16 + g * (16 * UNROLL)
        bbase = slot * WIN + g * (16 * UNROLL)
        for u in range(UNROLL):
            cursor = vec_body(ibase + u * 16, bbase + u * 16, cursor, parity)
            if (u + 1) % CHECK == 0:
                cursor, parity, nflush = _check_flush((cursor, parity, nflush))
        return (cursor, parity, nflush)

    def pair_body(i, carry):
        for slot in range(2):
            w = 2 * i + slot
            _win_wait()

            @pl.when(w + 1 < NWIN)
            def _(slot=slot):
                _win_dma(w + 1, 1 - slot)

            carry = lax.fori_loop(
                0, GPW, functools.partial(grp_body, slot), carry)
        return carry

    cursor, parity, nflush = lax.fori_loop(
        0, NWIN // 2, pair_body, (jnp.int32(0), jnp.int32(0), jnp.int32(0)))

    for si in range(2):
        @pl.when((parity == 1 - si) & (nflush >= 1))
        def _(si=si):
            _drain(si)

        @pl.when((parity == si) & (cursor > 0))
        def _(si=si):
            _fire(si)
            _drain(si)

    plsc.subcore_barrier()

    pltpu.sync_copy(acc_sum.at[pl.ds(off, CH_SLICE)], stage)
    pltpu.sync_copy(stage, part_hbm.at[pl.ds((c * 2 + 0) * NUM_CH_PAD + off, CH_SLICE)])
    pltpu.sync_copy(acc_cmb.at[pl.ds(off, CH_SLICE)], stage)
    pltpu.sync_copy(stage, part_hbm.at[pl.ds((c * 2 + 1) * NUM_CH_PAD + off, CH_SLICE)])


@jax.jit
def _sc_segsum(output, target, ids32):
    mesh = plsc.VectorSubcoreMesh(core_axis_name="c", subcore_axis_name="s")
    f = pl.kernel(
        _sc_body,
        out_type=jax.ShapeDtypeStruct((NC * 2 * NUM_CH_PAD,), jnp.float32),
        mesh=mesh,
        compiler_params=pltpu.CompilerParams(needs_layout_passes=False),
        scratch_types=[
            pltpu.VMEM((2 * WIN,), jnp.float32),
            pltpu.VMEM((2 * WIN,), jnp.float32),
            pltpu.VMEM((2 * IDW,), jnp.int32),
            pltpu.VMEM((CH_SLICE,), jnp.float32),
            pltpu.VMEM((2, LIST), jnp.int32),
            pltpu.VMEM((2, LIST), jnp.float32),
            pltpu.VMEM((2, LIST), jnp.float32),
            pltpu.VMEM_SHARED((NUM_CH_PAD,), jnp.float32),
            pltpu.VMEM_SHARED((NUM_CH_PAD,), jnp.float32),
            pltpu.SemaphoreType.DMA,
            pltpu.SemaphoreType.DMA,
            pltpu.SemaphoreType.DMA,
        ],
    )
    return f(output, target, ids32)


def _bce_body(p_ref, o_ref):
    st = p_ref[0, 0] + p_ref[1, 0]
    comb = p_ref[0, 1] + p_ref[1, 1]
    cnt = jnp.floor(comb * 0.5)
    tv = comb - 2.0 * cnt
    present = cnt > 0.0
    mean_p = st / jnp.maximum(cnt, 1.0)
    log_p = jnp.maximum(jnp.log(mean_p), -100.0)
    log_1mp = jnp.maximum(jnp.log(1.0 - mean_p), -100.0)
    per = -(tv * log_p + (1.0 - tv) * log_1mp)
    per = jnp.where(present, per, 0.0)
    n_present = jnp.maximum(jnp.sum(present.astype(jnp.float32)), 1.0)
    o_ref[0, 0] = jnp.sum(per) / n_present


@jax.jit
def _bce(partials):
    return pl.pallas_call(
        _bce_body,
        out_shape=jax.ShapeDtypeStruct((1, 1), jnp.float32),
        out_specs=pl.BlockSpec(memory_space=pltpu.SMEM),
    )(partials)


def kernel(output, target, ch_ids):
    ids32 = ch_ids.astype(jnp.int32)
    partials = _sc_segsum(output, target, ids32)
    loss = _bce(partials.reshape(NC, 2, NUM_CH_PAD // 128, 128))
    return loss[0, 0]


# speedup vs baseline: 2.3227x; 2.3227x over previous
import functools

import jax
import jax.numpy as jnp
from jax import lax
from jax.experimental import pallas as pl
from jax.experimental.pallas import tpu as pltpu
from jax.experimental.pallas import tpu_sc as plsc

N = 6400000
NUM_CH = 100000
NC = 2
NS = 16
NWK = NC * NS
PER_W = N // NWK
WIN = 4000
IDW = WIN + 32
NWIN = PER_W // WIN
UNROLL = 10
CHECK = 10
GPW = WIN // (16 * UNROLL)
LIST = 256
FLUSH_AT = LIST - 16 * CHECK
CH_SLICE = 6272
NUM_CH_PAD = NS * CH_SLICE


def _gather16(v, idx):
    return v.at[idx].get(mode="promise_in_bounds")


def _sc_body(out_hbm, tgt_hbm, ids_hbm, part_hbm,
             win_out, win_tgt, win_ids, stage,
             idx_l, sum_l, cmb_l,
             acc_sum, acc_cmb, dsem, fsem0, fsem1):
    c = lax.axis_index("c")
    s = lax.axis_index("s")
    wid = s * NC + c
    e0 = wid * PER_W
    io16 = lax.iota(jnp.int32, 16)
    io16p1 = io16 + 1
    m15 = io16 == 15
    mz = io16 > 0
    c15 = jnp.full((16,), 15, jnp.int32)
    pad_idx = NUM_CH + s * 16 + io16
    zero16 = jnp.zeros((16,), jnp.float32)
    fsems = (fsem0, fsem1)

    def _zb(i, _):
        stage[pl.ds(i * 16, 16)] = zero16
        return 0
    lax.fori_loop(0, CH_SLICE // 16, _zb, 0)
    off = s * CH_SLICE
    pltpu.sync_copy(stage, acc_sum.at[pl.ds(off, CH_SLICE)])
    pltpu.sync_copy(stage, acc_cmb.at[pl.ds(off, CH_SLICE)])

    def _fill(si):
        def _fb(k, _):
            for rr in range(2):
                idx_l[si, rr, pl.ds(16 * k, 16)] = pad_idx
                sum_l[si, rr, pl.ds(16 * k, 16)] = zero16
                cmb_l[si, rr, pl.ds(16 * k, 16)] = zero16
            return 0
        lax.fori_loop(0, 128 // 16, _fb, 0)

    _fill(0)
    _fill(1)

    def _fire(si):
        for rr in range(2):
            pltpu.async_copy(sum_l.at[si, rr], acc_sum.at[idx_l.at[si, rr]],
                             fsems[si], add=True)
            pltpu.async_copy(cmb_l.at[si, rr], acc_cmb.at[idx_l.at[si, rr]],
                             fsems[si], add=True)

    def _drain(si):
        for rr in range(2):
            pltpu.make_async_copy(sum_l.at[si, rr], acc_sum.at[idx_l.at[si, rr]],
                                  fsems[si]).wait()
            pltpu.make_async_copy(cmb_l.at[si, rr], acc_cmb.at[idx_l.at[si, rr]],
                                  fsems[si]).wait()

    def _win_dma(w, slot):
        base = e0 + w * WIN
        pltpu.async_copy(out_hbm.at[pl.ds(base, WIN)],
                         win_out.at[pl.ds(slot * WIN, WIN)], dsem)
        pltpu.async_copy(tgt_hbm.at[pl.ds(base, WIN)],
                         win_tgt.at[pl.ds(slot * WIN, WIN)], dsem)
        pltpu.async_copy(ids_hbm.at[pl.ds(base - 16, WIN + 16)],
                         win_ids.at[pl.ds(slot * IDW, WIN + 16)], dsem)

    def _win_wait():
        pltpu.make_async_copy(out_hbm.at[pl.ds(0, WIN)],
                              win_out.at[pl.ds(0, WIN)], dsem).wait()
        pltpu.make_async_copy(tgt_hbm.at[pl.ds(0, WIN)],
                              win_tgt.at[pl.ds(0, WIN)], dsem).wait()
        pltpu.make_async_copy(ids_hbm.at[pl.ds(0, WIN + 16)],
                              win_ids.at[pl.ds(0, WIN + 16)], dsem).wait()

    @pl.when(wid > 0)
    def _():
        pltpu.async_copy(ids_hbm.at[pl.ds(e0 - 16, WIN + 16)],
                         win_ids.at[pl.ds(0, WIN + 16)], dsem)

    @pl.when(wid == 0)
    def _():
        win_ids[pl.ds(0, 16)] = jnp.full((16,), -1, jnp.int32)
        pltpu.async_copy(ids_hbm.at[pl.ds(0, WIN + 16)],
                         win_ids.at[pl.ds(16, WIN + 16)], dsem)

    pltpu.async_copy(out_hbm.at[pl.ds(e0, WIN)],
                     win_out.at[pl.ds(0, WIN)], dsem)
    pltpu.async_copy(tgt_hbm.at[pl.ds(e0, WIN)],
                     win_tgt.at[pl.ds(0, WIN)], dsem)

    def vec_chunk(ioffs, boffs, curv, rowv):
        n = len(ioffs)
        ids = [win_ids[pl.ds(ioffs[k], 16)] for k in range(n)]
        sh = [win_ids[pl.ds(ioffs[k] - 1, 16)] for k in range(n)]
        shl = [win_ids[pl.ds(ioffs[k] + 1, 16)] for k in range(n)]
        x = [win_out[pl.ds(boffs[k], 16)] for k in range(n)]
        t = [win_tgt[pl.ds(boffs[k], 16)] for k in range(n)]
        p = [1.0 / (1.0 + jnp.exp(-x[k])) for k in range(n)]
        sb = [ids[k] != sh[k] for k in range(n)]
        sbz = [sb[k] & mz for k in range(n)]
        qz = [plsc.cumsum(jnp.where(sbz[k], 1, 0)) for k in range(n)]
        r = [plsc.cummax(jnp.where(sb[k], io16, 0)) for k in range(n)]
        cs = [plsc.cumsum(p[k]) for k in range(n)]
        csx = [cs[k] - p[k] for k in range(n)]
        base = [_gather16(csx[k], r[k]) for k in range(n)]
        runsum = [cs[k] - base[k] for k in range(n)]
        runcnt = [(io16p1 - r[k]).astype(jnp.float32) for k in range(n)]
        tsel = [jnp.where(sb[k], t[k], 0.0) for k in range(n)]
        tstart = [_gather16(tsel[k], r[k]) for k in range(n)]
        comb = [2.0 * runcnt[k] + tstart[k] for k in range(n)]
        eb = [(ids[k] != shl[k]) | m15 for k in range(n)]
        pos = []
        for k in range(n):
            pk = curv + qz[k]
            pos.append(pk)
            curv = _gather16(pk, c15) + 1
        for k in range(n):
            ph = jnp.right_shift(pos[k], 7)
            plo = pos[k] & 127
            plsc.store_scatter(idx_l, [rowv, ph, plo], ids[k], mask=eb[k])
            plsc.store_scatter(sum_l, [rowv, ph, plo], runsum[k], mask=eb[k])
            plsc.store_scatter(cmb_l, [rowv, ph, plo], comb[k], mask=eb[k])
        return curv

    def _check_flush(carry):
        curv, parity, nflush = carry
        do_flush = jnp.max(curv) >= FLUSH_AT

        @pl.when(do_flush)
        def _():
            for si in range(2):
                @pl.when(parity == si)
                def _(si=si):
                    _fire(si)

                    @pl.when(nflush >= 1)
                    def _():
                        _drain(1 - si)
                        _fill(1 - si)

        curv = jnp.where(do_flush, 0, curv)
        parity = jnp.where(do_flush, 1 - parity, parity)
        nflush = jnp.where(do_flush, nflush + 1, nflush)
        return (curv, parity, nflush)

    def grp_body(slot, g, carry):
        curv, parity, nflush = carry
        ibase = slot * IDW + 16 + g * (16 * UNROLL)
        bbase = slot * WIN + g * (16 * UNROLL)
        for u0 in range(0, UNROLL, CHECK):
            rowv = jnp.broadcast_to(parity, (16,))
            ioffs = [ibase + (u0 + k) * 16 for k in range(CHECK)]
            boffs = [bbase + (u0 + k) * 16 for k in range(CHECK)]
            curv = vec_chunk(ioffs, boffs, curv, rowv)
            curv, parity, nflush = _check_flush((curv, parity, nflush))
        return (curv, parity, nflush)

    def pair_body(i, carry):
        for slot in range(2):
            w = 2 * i + slot
            _win_wait()

            @pl.when(w + 1 < NWIN)
            def _(slot=slot):
                _win_dma(w + 1, 1 - slot)

            carry = lax.fori_loop(
                0, GPW, functools.partial(grp_body, slot), carry)
        return carry

    curv, parity, nflush = lax.fori_loop(
        0, NWIN // 2, pair_body,
        (jnp.zeros((16,), jnp.int32), jnp.int32(0), jnp.int32(0)))
    cursor = jnp.max(curv)

    for si in range(2):
        @pl.when((parity == 1 - si) & (nflush >= 1))
        def _(si=si):
            _drain(si)

        @pl.when((parity == si) & (cursor > 0))
        def _(si=si):
            _fire(si)
            _drain(si)

    plsc.subcore_barrier()

    pltpu.sync_copy(acc_sum.at[pl.ds(off, CH_SLICE)], stage)
    pltpu.sync_copy(stage, part_hbm.at[pl.ds((c * 2 + 0) * NUM_CH_PAD + off, CH_SLICE)])
    pltpu.sync_copy(acc_cmb.at[pl.ds(off, CH_SLICE)], stage)
    pltpu.sync_copy(stage, part_hbm.at[pl.ds((c * 2 + 1) * NUM_CH_PAD + off, CH_SLICE)])


@jax.jit
def _sc_segsum(output, target, ids32):
    mesh = plsc.VectorSubcoreMesh(core_axis_name="c", subcore_axis_name="s")
    f = pl.kernel(
        _sc_body,
        out_type=jax.ShapeDtypeStruct((NC * 2 * NUM_CH_PAD,), jnp.float32),
        mesh=mesh,
        compiler_params=pltpu.CompilerParams(needs_layout_passes=False),
        scratch_types=[
            pltpu.VMEM((2 * WIN,), jnp.float32),
            pltpu.VMEM((2 * WIN,), jnp.float32),
            pltpu.VMEM((2 * IDW,), jnp.int32),
            pltpu.VMEM((CH_SLICE,), jnp.float32),
            pltpu.VMEM((2, 2, 128), jnp.int32),
            pltpu.VMEM((2, 2, 128), jnp.float32),
            pltpu.VMEM((2, 2, 128), jnp.float32),
            pltpu.VMEM_SHARED((NUM_CH_PAD,), jnp.float32),
            pltpu.VMEM_SHARED((NUM_CH_PAD,), jnp.float32),
            pltpu.SemaphoreType.DMA,
            pltpu.SemaphoreType.DMA,
            pltpu.SemaphoreType.DMA,
        ],
    )
    return f(output, target, ids32)


def _bce_body(p_ref, o_ref):
    st = p_ref[0, 0] + p_ref[1, 0]
    comb = p_ref[0, 1] + p_ref[1, 1]
    cnt = jnp.floor(comb * 0.5)
    tv = comb - 2.0 * cnt
    present = cnt > 0.0
    mean_p = st / jnp.maximum(cnt, 1.0)
    log_p = jnp.maximum(jnp.log(mean_p), -100.0)
    log_1mp = jnp.maximum(jnp.log(1.0 - mean_p), -100.0)
    per = -(tv * log_p + (1.0 - tv) * log_1mp)
    per = jnp.where(present, per, 0.0)
    n_present = jnp.maximum(jnp.sum(present.astype(jnp.float32)), 1.0)
    o_ref[0, 0] = jnp.sum(per) / n_present


@jax.jit
def _bce(partials):
    return pl.pallas_call(
        _bce_body,
        out_shape=jax.ShapeDtypeStruct((1, 1), jnp.float32),
        out_specs=pl.BlockSpec(memory_space=pltpu.SMEM),
    )(partials)


def kernel(output, target, ch_ids):
    ids32 = ch_ids.astype(jnp.int32)
    partials = _sc_segsum(output, target, ids32)
    loss = _bce(partials.reshape(NC, 2, NUM_CH_PAD // 128, 128))
    return loss[0, 0]
